# baseline (speedup 1.0000x reference)
import jax
import jax.numpy as jnp
from jax.experimental import pallas as pl

_ROWS = 200000
_GRID = 25
_BR2 = _ROWS // 2 // _GRID
_BV = 2 * _ROWS // _GRID // 128


def _gate_mix_kernel(fp_ref, vp_ref, out_ref):
    f2 = fp_ref[...]
    sv = jnp.abs(vp_ref[0])
    spd = sv.reshape(_BV, 64, 2).sum(axis=2)
    sl = spd.reshape(_BR2, 2)
    fl = f2[:, :64]
    fr = f2[:, 64:]
    s_l = jnp.sum(jnp.abs(fl), axis=1, keepdims=True)
    s_r = jnp.sum(jnp.abs(fr), axis=1, keepdims=True)
    diff_l = s_l / jnp.maximum(s_l, 1e-6)
    diff_r = s_r / jnp.maximum(s_r, 1e-6)
    gate_l = jnp.exp(-diff_l) / (1.0 + 0.25 * sl[:, 0:1])
    gate_r = jnp.exp(-diff_r) / (1.0 + 0.25 * sl[:, 1:2])
    out_ref[:, :64] = (1.0 - gate_l) * fl
    out_ref[:, 64:] = (1.0 - gate_r) * fr


def _imap2(i):
    return i, jnp.int32(0)


def _imap3(i):
    return i, jnp.int32(0), jnp.int32(0)


def kernel(feats, vel_xy, coords):
    del coords
    n, c = feats.shape
    feats = feats.astype(jnp.float32)
    vel_xy = vel_xy.astype(jnp.float32)
    fp = feats.reshape(n // 2, 128)
    vp = vel_xy.reshape(_GRID, _BV, 128)
    out2 = pl.pallas_call(
        _gate_mix_kernel,
        grid=(_GRID,),
        in_specs=[
            pl.BlockSpec((_BR2, 128), _imap2),
            pl.BlockSpec((1, _BV, 128), _imap3),
        ],
        out_specs=pl.BlockSpec((_BR2, 128), _imap2),
        out_shape=jax.ShapeDtypeStruct((n // 2, 128), jnp.float32),
    )(fp, vp)
    return out2.reshape(n, c)


# speedup vs baseline: 1.9289x; 1.9289x over previous
import jax
import jax.numpy as jnp
from jax.experimental import pallas as pl
from jax.experimental.pallas import tpu as pltpu

_ROWS = 200000
_CH = 64
_BLOCK_ROWS = 8000


def _imap(i):
    return i, jnp.int32(0)


def _gate_mix_kernel(feats_ref, vel_ref, out_ref):
    f = feats_ref[...]
    v = vel_ref[...]
    speed = jnp.abs(v[:, 0]) + jnp.abs(v[:, 1])
    s = jnp.sum(jnp.abs(f), axis=1)
    diff = s / jnp.maximum(s, 1e-6)
    gate = jnp.exp(-diff) / (1.0 + 0.25 * speed)
    out_ref[...] = (1.0 - gate)[:, None] * f


def kernel(feats, vel_xy, coords):
    del coords
    n, c = feats.shape
    feats = feats.astype(jnp.float32)
    vel_xy = vel_xy.astype(jnp.float32)
    br = _BLOCK_ROWS if n == _ROWS else n
    grid = (n // br,)
    return pl.pallas_call(
        _gate_mix_kernel,
        grid=grid,
        in_specs=[
            pl.BlockSpec((br, c), _imap),
            pl.BlockSpec((br, 2), _imap),
        ],
        out_specs=pl.BlockSpec((br, c), _imap),
        out_shape=jax.ShapeDtypeStruct((n, c), jnp.float32),
        compiler_params=pltpu.CompilerParams(
            dimension_semantics=("parallel",)),
    )(feats, vel_xy)
